# baseline (speedup 1.0000x reference)
import jax
import jax.numpy as jnp
from jax.experimental import pallas as pl


def _add_kernel(x_ref, t_ref, o_ref):
    o_ref[...] = x_ref[...] + t_ref[...]


def kernel(x, table):
    B, L, D = x.shape
    BL = 256
    return pl.pallas_call(
        _add_kernel,
        grid=(L // BL, B),
        in_specs=[
            pl.BlockSpec((1, BL, D), lambda l, b: (b, l, 0)),
            pl.BlockSpec((BL, D), lambda l, b: (l, 0)),
        ],
        out_specs=pl.BlockSpec((1, BL, D), lambda l, b: (b, l, 0)),
        out_shape=jax.ShapeDtypeStruct(x.shape, x.dtype),
    )(x, table)


# speedup vs baseline: 1.4822x; 1.4822x over previous
import jax
import jax.numpy as jnp
from jax.experimental import pallas as pl


def _add_kernel(x_ref, t_ref, o_ref):
    o_ref[...] = x_ref[...] + t_ref[...]


def kernel(x, table):
    B, L, D = x.shape
    BL = 512
    return pl.pallas_call(
        _add_kernel,
        grid=(L // BL,),
        in_specs=[
            pl.BlockSpec((B, BL, D), lambda i: (0, i, 0)),
            pl.BlockSpec((BL, D), lambda i: (i, 0)),
        ],
        out_specs=pl.BlockSpec((B, BL, D), lambda i: (0, i, 0)),
        out_shape=jax.ShapeDtypeStruct(x.shape, x.dtype),
    )(x, table)
